# baseline (speedup 1.0000x reference)
import functools

import jax
import jax.numpy as jnp
from jax import lax
from jax.experimental import pallas as pl
from jax.experimental.pallas import tpu as pltpu
from jax.experimental.pallas import tpu_sc as plsc

BATCH = 16384
VOCAB = 100000
EMBED_DIM = 64
N_FEATS = 32
OUT_W = EMBED_DIM + N_FEATS
NC, NS, L = 2, 16, 16
NW = NC * NS
DPW = EMBED_DIM // NW
OCH = 4096
NCH = BATCH // OCH
SLICE = BATCH // NS
UNROLL = 16


def kernel(x, W):
  mesh = plsc.VectorSubcoreMesh(
      core_axis_name="c", subcore_axis_name="s", num_cores=NC, num_subcores=NS
  )

  @functools.partial(
      pl.kernel,
      out_type=jax.ShapeDtypeStruct((OUT_W, BATCH), jnp.float32),
      mesh=mesh,
      scratch_types=[
          pltpu.VMEM((VOCAB,), jnp.float32),
          pltpu.VMEM((BATCH,), jnp.int32),
          pltpu.VMEM((2, OCH), jnp.float32),
          pltpu.VMEM_SHARED((BATCH,), jnp.int32),
          pltpu.SemaphoreType.DMA,
          pltpu.SemaphoreType.DMA,
          pltpu.SemaphoreType.DMA,
          pltpu.SemaphoreType.DMA,
          pltpu.SemaphoreType.DMA,
      ],
      compiler_params=pltpu.CompilerParams(needs_layout_passes=False),
  )
  def k(xt_hbm, wt_hbm, out_hbm, w_row, ids_v, ob_v, ids_sh,
        sem_w, sem_ra, sem_rb, sem_wa, sem_wb):
    rsems = [sem_ra, sem_rb]
    wsems = [sem_wa, sem_wb]
    cid = lax.axis_index("c")
    sid = lax.axis_index("s")
    wid = sid * NC + cid
    d0 = wid * DPW

    w_copy = pltpu.async_copy(wt_hbm.at[d0], w_row, sem_w)

    pltpu.sync_copy(xt_hbm.at[0, pl.ds(sid * SLICE, SLICE)],
                    ob_v.at[0, pl.ds(0, SLICE)])

    @plsc.parallel_loop(0, SLICE, step=L, unroll=UNROLL)
    def conv(i):
      ids_v[pl.ds(i, L)] = ob_v[0, pl.ds(i, L)].astype(jnp.int32)

    pltpu.sync_copy(ids_v.at[pl.ds(0, SLICE)],
                    ids_sh.at[pl.ds(sid * SLICE, SLICE)])
    plsc.subcore_barrier()
    pltpu.sync_copy(ids_sh, ids_v)

    writes = {}
    for h in range(NCH):
      pltpu.async_copy(
          xt_hbm.at[1 + wid, pl.ds(h * OCH, OCH)], ob_v.at[h % 2],
          rsems[h % 2]).wait()
      writes[h] = pltpu.async_copy(
          ob_v.at[h % 2], out_hbm.at[EMBED_DIM + wid, pl.ds(h * OCH, OCH)],
          wsems[h % 2])
      if h >= 1:
        writes.pop(h - 1).wait()
    writes.pop(NCH - 1).wait()

    for t in range(DPW):
      d = d0 + t
      w_copy.wait()
      for h in range(NCH):
        g = t * NCH + h
        if g >= 2:
          writes.pop(g - 2).wait()

        @plsc.parallel_loop(0, OCH, step=L, unroll=UNROLL)
        def body(i):
          idx = ids_v[pl.ds(h * OCH + i, L)]
          ob_v[g % 2, pl.ds(i, L)] = plsc.load_gather(w_row, [idx])

        if h == NCH - 1 and t + 1 < DPW:
          w_copy = pltpu.async_copy(wt_hbm.at[d + 1], w_row, sem_w)
        writes[g] = pltpu.async_copy(
            ob_v.at[g % 2], out_hbm.at[d, pl.ds(h * OCH, OCH)], wsems[g % 2])
    writes.pop(DPW * NCH - 2).wait()
    writes.pop(DPW * NCH - 1).wait()

  out_t = k(x.T, W.T)
  return out_t.T


# speedup vs baseline: 1.0053x; 1.0053x over previous
import functools

import jax
import jax.numpy as jnp
from jax import lax
from jax.experimental import pallas as pl
from jax.experimental.pallas import tpu as pltpu
from jax.experimental.pallas import tpu_sc as plsc

BATCH = 16384
VOCAB = 100000
EMBED_DIM = 64
N_FEATS = 32
OUT_W = EMBED_DIM + N_FEATS
NC, NS, L = 2, 16, 16
NW = NC * NS
DPW = EMBED_DIM // NW
OCH = 4096
NCH = BATCH // OCH
SLICE = BATCH // NS
UNROLL = 8


def kernel(x, W):
  mesh = plsc.VectorSubcoreMesh(
      core_axis_name="c", subcore_axis_name="s", num_cores=NC, num_subcores=NS
  )

  @functools.partial(
      pl.kernel,
      out_type=jax.ShapeDtypeStruct((OUT_W, BATCH), jnp.float32),
      mesh=mesh,
      scratch_types=[
          pltpu.VMEM((VOCAB,), jnp.float32),
          pltpu.VMEM((BATCH,), jnp.int32),
          pltpu.VMEM((2, OCH), jnp.float32),
          pltpu.VMEM_SHARED((BATCH,), jnp.int32),
          pltpu.SemaphoreType.DMA,
          pltpu.SemaphoreType.DMA,
          pltpu.SemaphoreType.DMA,
          pltpu.SemaphoreType.DMA,
          pltpu.SemaphoreType.DMA,
      ],
      compiler_params=pltpu.CompilerParams(needs_layout_passes=False),
  )
  def k(xt_hbm, wt_hbm, out_hbm, w_row, ids_v, ob_v, ids_sh,
        sem_w, sem_ra, sem_rb, sem_wa, sem_wb):
    rsems = [sem_ra, sem_rb]
    wsems = [sem_wa, sem_wb]
    cid = lax.axis_index("c")
    sid = lax.axis_index("s")
    wid = sid * NC + cid
    d0 = wid * DPW

    w_copy = pltpu.async_copy(wt_hbm.at[d0], w_row, sem_w)

    pltpu.sync_copy(xt_hbm.at[0, pl.ds(sid * SLICE, SLICE)],
                    ob_v.at[0, pl.ds(0, SLICE)])

    @plsc.parallel_loop(0, SLICE, step=L, unroll=UNROLL)
    def conv(i):
      ids_v[pl.ds(i, L)] = ob_v[0, pl.ds(i, L)].astype(jnp.int32)

    pltpu.sync_copy(ids_v.at[pl.ds(0, SLICE)],
                    ids_sh.at[pl.ds(sid * SLICE, SLICE)])

    writes = {}
    for h in range(NCH):
      pltpu.async_copy(
          xt_hbm.at[1 + wid, pl.ds(h * OCH, OCH)], ob_v.at[h % 2],
          rsems[h % 2]).wait()
      writes[h] = pltpu.async_copy(
          ob_v.at[h % 2], out_hbm.at[EMBED_DIM + wid, pl.ds(h * OCH, OCH)],
          wsems[h % 2])
      if h >= 1:
        writes.pop(h - 1).wait()
    writes.pop(NCH - 1).wait()

    plsc.subcore_barrier()
    pltpu.sync_copy(ids_sh, ids_v)

    for t in range(DPW):
      d = d0 + t
      w_copy.wait()
      for h in range(NCH):
        g = t * NCH + h
        if g >= 2:
          writes.pop(g - 2).wait()

        @plsc.parallel_loop(0, OCH, step=L, unroll=UNROLL)
        def body(i):
          idx = ids_v[pl.ds(h * OCH + i, L)]
          ob_v[g % 2, pl.ds(i, L)] = plsc.load_gather(w_row, [idx])

        if h == NCH - 1 and t + 1 < DPW:
          w_copy = pltpu.async_copy(wt_hbm.at[d + 1], w_row, sem_w)
        writes[g] = pltpu.async_copy(
            ob_v.at[g % 2], out_hbm.at[d, pl.ds(h * OCH, OCH)], wsems[g % 2])
    writes.pop(DPW * NCH - 2).wait()
    writes.pop(DPW * NCH - 1).wait()

  out_t = k(x.T, W.T)
  return out_t.T


# speedup vs baseline: 1.0174x; 1.0121x over previous
import functools

import jax
import jax.numpy as jnp
from jax import lax
from jax.experimental import pallas as pl
from jax.experimental.pallas import tpu as pltpu
from jax.experimental.pallas import tpu_sc as plsc

BATCH = 16384
VOCAB = 100000
EMBED_DIM = 64
N_FEATS = 32
OUT_W = EMBED_DIM + N_FEATS
NC, NS, L = 2, 16, 16
NW = NC * NS
DPW = EMBED_DIM // NW
OCH = 4096
NCH = BATCH // OCH
SLICE = BATCH // NS
UNROLL = 8


def kernel(x, W):
  mesh = plsc.VectorSubcoreMesh(
      core_axis_name="c", subcore_axis_name="s", num_cores=NC, num_subcores=NS
  )

  @functools.partial(
      pl.kernel,
      out_type=jax.ShapeDtypeStruct((OUT_W, BATCH), jnp.float32),
      mesh=mesh,
      scratch_types=[
          pltpu.VMEM((VOCAB,), jnp.float32),
          pltpu.VMEM((BATCH,), jnp.int32),
          pltpu.VMEM((2, OCH), jnp.float32),
          pltpu.VMEM_SHARED((BATCH,), jnp.int32),
          pltpu.SemaphoreType.DMA,
          pltpu.SemaphoreType.DMA,
          pltpu.SemaphoreType.DMA,
          pltpu.SemaphoreType.DMA,
          pltpu.SemaphoreType.DMA,
      ],
      compiler_params=pltpu.CompilerParams(needs_layout_passes=False),
  )
  def k(xt_hbm, wt_hbm, out_hbm, w_row, ids_v, ob_v, ids_sh,
        sem_w, sem_ra, sem_rb, sem_wa, sem_wb):
    rsems = [sem_ra, sem_rb]
    wsems = [sem_wa, sem_wb]
    cid = lax.axis_index("c")
    sid = lax.axis_index("s")
    wid = sid * NC + cid
    d0 = wid * DPW

    w_copy = pltpu.async_copy(wt_hbm.at[d0], w_row, sem_w)

    pltpu.sync_copy(xt_hbm.at[0, pl.ds(sid * SLICE, SLICE)],
                    ob_v.at[0, pl.ds(0, SLICE)])

    @plsc.parallel_loop(0, SLICE, step=L, unroll=UNROLL)
    def conv(i):
      ids_v[pl.ds(i, L)] = ob_v[0, pl.ds(i, L)].astype(jnp.int32)

    pltpu.sync_copy(ids_v.at[pl.ds(0, SLICE)],
                    ids_sh.at[pl.ds(sid * SLICE, SLICE)])
    plsc.subcore_barrier()
    pltpu.sync_copy(ids_sh, ids_v)

    writes = {}
    for h in range(NCH):
      pltpu.async_copy(
          xt_hbm.at[1 + wid, pl.ds(h * OCH, OCH)], ob_v.at[h % 2],
          rsems[h % 2]).wait()
      writes[h] = pltpu.async_copy(
          ob_v.at[h % 2], out_hbm.at[EMBED_DIM + wid, pl.ds(h * OCH, OCH)],
          wsems[h % 2])
      if h >= 1:
        writes.pop(h - 1).wait()
    writes.pop(NCH - 1).wait()

    for t in range(DPW):
      d = d0 + t
      w_copy.wait()
      for h in range(NCH):
        g = t * NCH + h
        if g >= 2:
          writes.pop(g - 2).wait()

        @plsc.parallel_loop(0, OCH, step=L, unroll=UNROLL)
        def body(i):
          idx = ids_v[pl.ds(h * OCH + i, L)]
          ob_v[g % 2, pl.ds(i, L)] = plsc.load_gather(w_row, [idx])

        if h == NCH - 1 and t + 1 < DPW:
          w_copy = pltpu.async_copy(wt_hbm.at[d + 1], w_row, sem_w)
        writes[g] = pltpu.async_copy(
            ob_v.at[g % 2], out_hbm.at[d, pl.ds(h * OCH, OCH)], wsems[g % 2])
    writes.pop(DPW * NCH - 2).wait()
    writes.pop(DPW * NCH - 1).wait()

  out_t = k(x.T, W.T)
  return out_t.T
